# baseline (speedup 1.0000x reference)
import functools

import jax
import jax.numpy as jnp
from jax import lax
from jax.experimental import pallas as pl
from jax.experimental.pallas import tpu as pltpu
from jax.experimental.pallas import tpu_sc as plsc

N = 10000
E = 320000
H = 128
HH = 64
C = 112
EPS = 1e-7

NC = 2
NT = 16
B = 128
CHUNKS = 160
E_PAD = NT * B * CHUNKS
NH = N // 2
N_ACC = 5120
ROWS_PT = N_ACC // NT
JUNK = 5100

EH = E_PAD // 2
RB = 200
GRID_N = N // RB


@functools.cache
def _build_sc_edge_pass():
    mesh = plsc.VectorSubcoreMesh(
        core_axis_name="c", subcore_axis_name="s", num_cores=NC,
        num_subcores=NT)

    @functools.partial(
        pl.kernel,
        out_type=jax.ShapeDtypeStruct((NC, 2, N_ACC, H), jnp.float32),
        mesh=mesh,
        scratch_types=[
            pltpu.VMEM((4, B), jnp.int32),
            pltpu.VMEM((B, H), jnp.float32),
            pltpu.VMEM((B, H), jnp.float32),
            pltpu.VMEM((B // 2, H), jnp.float32),
            pltpu.VMEM((B // 2, H), jnp.float32),
            pltpu.VMEM((B, H), jnp.float32),
            pltpu.VMEM((8, B), jnp.int32),
            pltpu.VMEM((HH,), jnp.float32),
            pltpu.VMEM((HH,), jnp.float32),
            pltpu.VMEM((16,), jnp.float32),
            pltpu.VMEM_SHARED((N_ACC, H), jnp.float32),
            pltpu.SemaphoreType.DMA,
            pltpu.SemaphoreType.DMA,
            pltpu.SemaphoreType.DMA,
            pltpu.SemaphoreType.DMA,
            pltpu.SemaphoreType.DMA,
            pltpu.SemaphoreType.DMA,
        ],
    )
    def k(z_h, ea0_h, ea1_h, src_h, dst_h, zmax_h, eamax_h, tvec_h, out_h,
          idxs, zb_a, zb_b, ea_a, ea_b, cbuf, idxb, zmv, emv, tv,
          acc, sem_za, sem_zb, sem_ea, sem_eb, sem_ia, sem_ib):
        c = lax.axis_index("c")
        s = lax.axis_index("s")
        zrow = s * ROWS_PT

        pltpu.sync_copy(zmax_h.at[pl.ds(c * HH, HH)], zmv)
        pltpu.sync_copy(eamax_h.at[pl.ds(c * HH, HH)], emv)
        pltpu.sync_copy(tvec_h, tv)
        treg = tv[...]
        lbs = []
        for f in range(HH // 16):
            sl = pl.ds(f * 16, 16)
            m = jnp.maximum(zmv[sl] + emv[sl], 0.0) + EPS
            lbs.append(jnp.maximum(m * treg, EPS * treg) - EPS * treg)

        cN = c * N
        ea_base = s * CHUNKS * (B // 2)

        for hp in range(2):
            @pl.loop(0, B)
            def _zero(r):
                for f in range(H // 16):
                    cbuf[r, pl.ds(f * 16, 16)] = jnp.zeros((16,), jnp.float32)

            for kk in range(ROWS_PT // B):
                pltpu.sync_copy(cbuf, acc.at[pl.ds(zrow + kk * B, B)])
            rem = ROWS_PT % B
            if rem:
                pltpu.sync_copy(cbuf.at[pl.ds(0, rem)],
                                acc.at[pl.ds(zrow + (ROWS_PT // B) * B, rem)])

            plsc.subcore_barrier()

            def issue_idx(j, half, si):
                base = (s * CHUNKS + j) * B
                pltpu.async_copy(src_h.at[pl.ds(base, B)],
                                 idxs.at[2 * half], si)
                pltpu.async_copy(dst_h.at[pl.ds(base, B)],
                                 idxs.at[2 * half + 1], si)

            def issue_gather(j, half, zb, eb, sz, se, si):
                pltpu.make_async_copy(src_h.at[pl.ds(0, B)],
                                      idxs.at[0], si).wait()
                pltpu.make_async_copy(src_h.at[pl.ds(0, B)],
                                      idxs.at[0], si).wait()
                for f in range(B // 16):
                    sl = pl.ds(f * 16, 16)
                    idxs[2 * half, sl] = idxs[2 * half, sl] + cN
                pltpu.async_copy(z_h.at[idxs.at[2 * half]], zb, sz)
                erows = pl.ds(ea_base + j * (B // 2), B // 2)

                @pl.when(c == 0)
                def _():
                    pltpu.async_copy(ea0_h.at[erows], eb, se)

                @pl.when(c == 1)
                def _():
                    pltpu.async_copy(ea1_h.at[erows], eb, se)

            def do_chunk(half, zb, eb, sz, se):
                for f in range(B // 16):
                    sl = pl.ds(f * 16, 16)
                    loc = idxs[2 * half + 1, sl] - hp * NH
                    ok = (loc >= 0) & (loc < NH)
                    idxb[half, sl] = jnp.where(ok, loc, JUNK)
                pltpu.make_async_copy(z_h.at[idxs.at[2 * half]],
                                      zb, sz).wait()
                pltpu.make_async_copy(
                    ea0_h.at[pl.ds(0, B // 2)], eb, se).wait()

                @pl.loop(0, B // 2)
                def _pair(pr):
                    for par in range(2):
                        r = par * (B // 2) + pr
                        for f in range(HH // 16):
                            zsl = pl.ds(f * 16, 16)
                            esl = pl.ds(par * HH + f * 16, 16)
                            m0 = jnp.maximum(
                                zb[r, zsl] + eb[pr, esl], 0.0)
                            ex = jnp.exp(m0 * treg - lbs[f])
                            cbuf[r, pl.ds(f * 16, 16)] = ex
                            cbuf[r, pl.ds(HH + f * 16, 16)] = m0 * ex

            def do_scatter(half):
                pltpu.sync_copy(cbuf, acc.at[idxb.at[half]], add=True)

            issue_idx(0, 0, sem_ia)
            issue_gather(0, 0, zb_a, ea_a, sem_za, sem_ea, sem_ia)
            issue_idx(1, 1, sem_ib)
            issue_gather(1, 1, zb_b, ea_b, sem_zb, sem_eb, sem_ib)

            @pl.loop(0, CHUNKS, step=2)
            def _chunk2(j):
                do_chunk(0, zb_a, ea_a, sem_za, sem_ea)

                @pl.when(j + 2 < CHUNKS)
                def _():
                    issue_idx(j + 2, 0, sem_ia)
                    issue_gather(j + 2, 0, zb_a, ea_a, sem_za, sem_ea,
                                 sem_ia)

                do_scatter(0)
                do_chunk(1, zb_b, ea_b, sem_zb, sem_eb)

                @pl.when(j + 3 < CHUNKS)
                def _():
                    issue_idx(j + 3, 1, sem_ib)
                    issue_gather(j + 3, 1, zb_b, ea_b, sem_zb, sem_eb,
                                 sem_ib)

                do_scatter(1)

            plsc.subcore_barrier()

            pltpu.sync_copy(acc.at[pl.ds(zrow, ROWS_PT)],
                            out_h.at[c, hp, pl.ds(zrow, ROWS_PT)])

            if hp == 0:
                plsc.subcore_barrier()

    return k


def _sc_edge_pass(z, ea_p0, ea_p1, src2d, dst2d, zmaxv, eamaxv, tvec):
    return _build_sc_edge_pass()(z, ea_p0, ea_p1, src2d, dst2d, zmaxv,
                                 eamaxv, tvec)


def _node_proj(x, w, b):
    def body(x_r, w_r, b_r, z_r, m_r):
        i = pl.program_id(0)
        h = jnp.dot(x_r[...], w_r[...],
                    preferred_element_type=jnp.float32) + b_r[...]
        z_r[0] = h
        z_r[1] = jnp.concatenate([h[:, HH:], h[:, :HH]], axis=-1)
        m = jnp.max(h, axis=0, keepdims=True)

        @pl.when(i == 0)
        def _():
            m_r[...] = m

        @pl.when(i != 0)
        def _():
            m_r[...] = jnp.maximum(m_r[...], m)

    return pl.pallas_call(
        body,
        grid=(GRID_N,),
        in_specs=[
            pl.BlockSpec((RB, H), lambda i: (i, 0)),
            pl.BlockSpec((H, H), lambda i: (0, 0)),
            pl.BlockSpec((1, H), lambda i: (0, 0)),
        ],
        out_specs=[
            pl.BlockSpec((2, RB, H), lambda i: (0, i, 0)),
            pl.BlockSpec((1, H), lambda i: (0, 0)),
        ],
        out_shape=[
            jax.ShapeDtypeStruct((2, N, H), jnp.float32),
            jax.ShapeDtypeStruct((1, H), jnp.float32),
        ],
    )(x, w, b)


def _edge_proj(eap, w, b, c0):
    EB = 2048
    grid = EH // EB

    def body(a_lo, a_hi, w_r, b_r, e_r, m_r):
        i = pl.program_id(0)
        wv = w_r[...]
        bv = b_r[...]
        lo = jnp.dot(a_lo[...], wv, preferred_element_type=jnp.float32) + bv
        hi = jnp.dot(a_hi[...], wv, preferred_element_type=jnp.float32) + bv
        e_r[...] = jnp.concatenate(
            [lo[:, c0:c0 + HH], hi[:, c0:c0 + HH]], axis=-1)
        m = jnp.maximum(jnp.max(lo, axis=0, keepdims=True),
                        jnp.max(hi, axis=0, keepdims=True))

        @pl.when(i == 0)
        def _():
            m_r[...] = m

        @pl.when(i != 0)
        def _():
            m_r[...] = jnp.maximum(m_r[...], m)

    return pl.pallas_call(
        body,
        grid=(grid,),
        in_specs=[
            pl.BlockSpec((EB, 16), lambda i: (i, 0)),
            pl.BlockSpec((EB, 16), lambda i: (i + EH // EB, 0)),
            pl.BlockSpec((16, H), lambda i: (0, 0)),
            pl.BlockSpec((1, H), lambda i: (0, 0)),
        ],
        out_specs=[
            pl.BlockSpec((EB, H), lambda i: (i, 0)),
            pl.BlockSpec((1, H), lambda i: (0, 0)),
        ],
        out_shape=[
            jax.ShapeDtypeStruct((EH, H), jnp.float32),
            jax.ShapeDtypeStruct((1, H), jnp.float32),
        ],
    )(eap, eap, w, b)


def _layer_norm_blk(u, g, b):
    m = jnp.mean(u, axis=-1, keepdims=True)
    v = jnp.mean((u - m) ** 2, axis=-1, keepdims=True)
    return (u - m) / jnp.sqrt(v + 1e-5) * g + b


def _combine_mlp(comb, z, h_prev, w1, b1, mg, mb, w2, b2, nlg, nlb):
    def body(lo_r, hi_r, z_r, hp_r, w1_r, b1_r, mg_r, mb_r, w2_r, b2_r,
             g_r, bb_r, h_r, zn_r, m_r):
        lo = lo_r[0, 0]
        hi = hi_r[0, 0]
        den = jnp.concatenate([lo[:, :HH], hi[:, :HH]], axis=-1)
        num = jnp.concatenate([lo[:, HH:], hi[:, HH:]], axis=-1) + EPS * den
        out = num / (den + 1e-16) + z_r[0]
        u = jnp.dot(out, w1_r[...], preferred_element_type=jnp.float32)
        u = _layer_norm_blk(u + b1_r[...], mg_r[...], mb_r[...])
        u = jnp.maximum(u, 0.0)
        v = jnp.dot(u, w2_r[...], preferred_element_type=jnp.float32)
        h = hp_r[...] + v + b2_r[...]
        zn = jnp.maximum(_layer_norm_blk(h, g_r[...], bb_r[...]), 0.0)
        h_r[...] = h
        zn_r[0] = zn
        zn_r[1] = jnp.concatenate([zn[:, HH:], zn[:, :HH]], axis=-1)
        i = pl.program_id(0)
        m = jnp.max(zn, axis=0, keepdims=True)

        @pl.when(i == 0)
        def _():
            m_r[...] = m

        @pl.when(i != 0)
        def _():
            m_r[...] = jnp.maximum(m_r[...], m)

    BPH = NH // RB

    in_specs = [
        pl.BlockSpec((1, 1, RB, H),
                     lambda i: (0, i // BPH, i % BPH, 0)),
        pl.BlockSpec((1, 1, RB, H),
                     lambda i: (1, i // BPH, i % BPH, 0)),
        pl.BlockSpec((1, RB, H), lambda i: (0, i, 0)),
        pl.BlockSpec((RB, H), lambda i: (i, 0)),
        pl.BlockSpec((H, 2 * H), lambda i: (0, 0)),
        pl.BlockSpec((1, 2 * H), lambda i: (0, 0)),
        pl.BlockSpec((1, 2 * H), lambda i: (0, 0)),
        pl.BlockSpec((1, 2 * H), lambda i: (0, 0)),
        pl.BlockSpec((2 * H, H), lambda i: (0, 0)),
        pl.BlockSpec((1, H), lambda i: (0, 0)),
        pl.BlockSpec((1, H), lambda i: (0, 0)),
        pl.BlockSpec((1, H), lambda i: (0, 0)),
    ]
    out_specs = [
        pl.BlockSpec((RB, H), lambda i: (i, 0)),
        pl.BlockSpec((2, RB, H), lambda i: (0, i, 0)),
        pl.BlockSpec((1, H), lambda i: (0, 0)),
    ]
    out_shape = [
        jax.ShapeDtypeStruct((N, H), jnp.float32),
        jax.ShapeDtypeStruct((2, N, H), jnp.float32),
        jax.ShapeDtypeStruct((1, H), jnp.float32),
    ]
    return pl.pallas_call(
        body,
        grid=(GRID_N,),
        in_specs=in_specs,
        out_specs=out_specs,
        out_shape=out_shape,
    )(comb, comb, z, h_prev, w1, b1.reshape(1, -1), mg.reshape(1, -1),
      mb.reshape(1, -1), w2, b2.reshape(1, -1), nlg.reshape(1, -1),
      nlb.reshape(1, -1))


def _head(z, w, b):
    def body(z_r, w_r, b_r, y_r):
        y_r[...] = jnp.dot(z_r[0], w_r[...],
                           preferred_element_type=jnp.float32) + b_r[...]

    return pl.pallas_call(
        body,
        grid=(GRID_N,),
        in_specs=[
            pl.BlockSpec((1, RB, H), lambda i: (0, i, 0)),
            pl.BlockSpec((H, C), lambda i: (0, 0)),
            pl.BlockSpec((1, C), lambda i: (0, 0)),
        ],
        out_specs=pl.BlockSpec((RB, C), lambda i: (i, 0)),
        out_shape=jax.ShapeDtypeStruct((N, C), jnp.float32),
    )(z, w, b.reshape(1, -1))


def kernel(x, edge_index, edge_attr, params):
    src = edge_index[0].astype(jnp.int32)
    dst = edge_index[1].astype(jnp.int32)
    pad = E_PAD - E
    src_p = jnp.concatenate([src, jnp.zeros((pad,), jnp.int32)])
    dst_p = jnp.concatenate([dst, jnp.full((pad,), N, jnp.int32)])
    src2d = jnp.concatenate([src_p[:EH].reshape(-1, B // 2),
                             src_p[EH:].reshape(-1, B // 2)],
                            axis=1).reshape(E_PAD)
    dst2d = jnp.concatenate([dst_p[:EH].reshape(-1, B // 2),
                             dst_p[EH:].reshape(-1, B // 2)],
                            axis=1).reshape(E_PAD)
    eap = jnp.concatenate(
        [edge_attr, jnp.zeros((pad, edge_attr.shape[1]), jnp.float32)])

    ea_p0, eamax = _edge_proj(eap, params['W_edge'],
                              params['b_edge'].reshape(1, -1), 0)
    ea_p1, _ = _edge_proj(eap, params['W_edge'],
                          params['b_edge'].reshape(1, -1), HH)
    eamaxv = eamax.reshape(H)

    z, zmax = _node_proj(x, params['W_node'], params['b_node'].reshape(1, -1))

    ls = params['layers']
    nxt_g = jnp.stack([ls[1]['ln_g'], ls[2]['ln_g'], ls[0]['ln_g']])
    nxt_b = jnp.stack([ls[1]['ln_b'], ls[2]['ln_b'], ls[0]['ln_b']])
    xs = {
        't': jnp.stack([p['t'] for p in ls]),
        'W1': jnp.stack([p['W1'] for p in ls]),
        'b1': jnp.stack([p['b1'] for p in ls]),
        'mg': jnp.stack([p['mln_g'] for p in ls]),
        'mb': jnp.stack([p['mln_b'] for p in ls]),
        'W2': jnp.stack([p['W2'] for p in ls]),
        'b2': jnp.stack([p['b2'] for p in ls]),
        'ng': nxt_g,
        'nb': nxt_b,
    }

    def step(carry, p):
        h, zc, zm = carry
        tvec = jnp.full((16,), p['t'], jnp.float32)
        comb = _sc_edge_pass(zc.reshape(2 * N, H), ea_p0, ea_p1, src2d,
                             dst2d, zm.reshape(H), eamaxv, tvec)
        h2, z2, zm2 = _combine_mlp(comb, zc, h, p['W1'], p['b1'], p['mg'],
                                   p['mb'], p['W2'], p['b2'], p['ng'],
                                   p['nb'])
        return (h2, z2, zm2), None

    h0 = jnp.zeros((N, H), jnp.float32)
    (h, z, zmax), _ = jax.lax.scan(step, (h0, z, zmax), xs)
    return _head(z, params['W_lin'], params['b_lin'])


# speedup vs baseline: 1.0071x; 1.0071x over previous
import functools

import jax
import jax.numpy as jnp
from jax import lax
from jax.experimental import pallas as pl
from jax.experimental.pallas import tpu as pltpu
from jax.experimental.pallas import tpu_sc as plsc

N = 10000
E = 320000
H = 128
HH = 64
C = 112
EPS = 1e-7

NC = 2
NT = 16
B = 128
CHUNKS = 160
E_PAD = NT * B * CHUNKS
NH = N // 2
N_ACC = 5120
ROWS_PT = N_ACC // NT
JUNK = 5100

EH = E_PAD // 2
RB = 200
GRID_N = N // RB


@functools.cache
def _build_sc_edge_pass():
    mesh = plsc.VectorSubcoreMesh(
        core_axis_name="c", subcore_axis_name="s", num_cores=NC,
        num_subcores=NT)

    @functools.partial(
        pl.kernel,
        out_type=jax.ShapeDtypeStruct((NC, 2, N_ACC, H), jnp.float32),
        mesh=mesh,
        scratch_types=[
            pltpu.VMEM((4, B), jnp.int32),
            pltpu.VMEM((B, H), jnp.float32),
            pltpu.VMEM((B, H), jnp.float32),
            pltpu.VMEM((B // 2, H), jnp.float32),
            pltpu.VMEM((B // 2, H), jnp.float32),
            pltpu.VMEM((B, H), jnp.float32),
            pltpu.VMEM((8, B), jnp.int32),
            pltpu.VMEM((HH,), jnp.float32),
            pltpu.VMEM((HH,), jnp.float32),
            pltpu.VMEM((16,), jnp.float32),
            pltpu.VMEM_SHARED((N_ACC, H), jnp.float32),
            pltpu.SemaphoreType.DMA,
            pltpu.SemaphoreType.DMA,
            pltpu.SemaphoreType.DMA,
            pltpu.SemaphoreType.DMA,
            pltpu.SemaphoreType.DMA,
            pltpu.SemaphoreType.DMA,
        ],
    )
    def k(z_h, ea0_h, ea1_h, src_h, dst_h, zmax_h, eamax_h, tvec_h, out_h,
          idxs, zb_a, zb_b, ea_a, ea_b, cbuf, idxb, zmv, emv, tv,
          acc, sem_za, sem_zb, sem_ea, sem_eb, sem_ia, sem_ib):
        c = lax.axis_index("c")
        s = lax.axis_index("s")
        zrow = s * ROWS_PT

        pltpu.sync_copy(zmax_h.at[pl.ds(c * HH, HH)], zmv)
        pltpu.sync_copy(eamax_h.at[pl.ds(c * HH, HH)], emv)
        pltpu.sync_copy(tvec_h, tv)
        treg = tv[...]
        lbs = []
        for f in range(HH // 16):
            sl = pl.ds(f * 16, 16)
            m = jnp.maximum(zmv[sl] + emv[sl], 0.0) + EPS
            lbs.append(jnp.maximum(m * treg, EPS * treg) - EPS * treg)

        cN = c * N
        ea_base = s * CHUNKS * (B // 2)

        for hp in range(2):
            @pl.loop(0, B)
            def _zero(r):
                for f in range(H // 16):
                    cbuf[r, pl.ds(f * 16, 16)] = jnp.zeros((16,), jnp.float32)

            for kk in range(ROWS_PT // B):
                pltpu.sync_copy(cbuf, acc.at[pl.ds(zrow + kk * B, B)])
            rem = ROWS_PT % B
            if rem:
                pltpu.sync_copy(cbuf.at[pl.ds(0, rem)],
                                acc.at[pl.ds(zrow + (ROWS_PT // B) * B, rem)])

            plsc.subcore_barrier()

            def issue_idx(j, half, si):
                base = (s * CHUNKS + j) * B
                pltpu.async_copy(src_h.at[pl.ds(base, B)],
                                 idxs.at[2 * half], si)
                pltpu.async_copy(dst_h.at[pl.ds(base, B)],
                                 idxs.at[2 * half + 1], si)

            def issue_gather(j, half, zb, eb, sz, se, si):
                pltpu.make_async_copy(src_h.at[pl.ds(0, B)],
                                      idxs.at[0], si).wait()
                pltpu.make_async_copy(src_h.at[pl.ds(0, B)],
                                      idxs.at[0], si).wait()
                for f in range(B // 16):
                    sl = pl.ds(f * 16, 16)
                    idxs[2 * half, sl] = idxs[2 * half, sl] + cN
                pltpu.async_copy(z_h.at[idxs.at[2 * half]], zb, sz)
                erows = pl.ds(ea_base + j * (B // 2), B // 2)

                @pl.when(c == 0)
                def _():
                    pltpu.async_copy(ea0_h.at[erows], eb, se)

                @pl.when(c == 1)
                def _():
                    pltpu.async_copy(ea1_h.at[erows], eb, se)

            def do_chunk(half, zb, eb, sz, se):
                for f in range(B // 16):
                    sl = pl.ds(f * 16, 16)
                    loc = idxs[2 * half + 1, sl] - hp * NH
                    ok = (loc >= 0) & (loc < NH)
                    idxb[half, sl] = jnp.where(ok, loc, JUNK)
                pltpu.make_async_copy(z_h.at[idxs.at[2 * half]],
                                      zb, sz).wait()
                pltpu.make_async_copy(
                    ea0_h.at[pl.ds(0, B // 2)], eb, se).wait()

                @pl.loop(0, B // 2)
                def _pair(pr):
                    for par in range(2):
                        r = par * (B // 2) + pr
                        for f in range(HH // 16):
                            zsl = pl.ds(f * 16, 16)
                            esl = pl.ds(par * HH + f * 16, 16)
                            m0 = jnp.maximum(
                                zb[r, zsl] + eb[pr, esl], 0.0)
                            ex = jnp.exp(m0 * treg - lbs[f])
                            cbuf[r, pl.ds(f * 16, 16)] = ex
                            cbuf[r, pl.ds(HH + f * 16, 16)] = m0 * ex

                pltpu.sync_copy(cbuf, acc.at[idxb.at[half]], add=True)

            issue_idx(0, 0, sem_ia)
            issue_gather(0, 0, zb_a, ea_a, sem_za, sem_ea, sem_ia)
            issue_idx(1, 1, sem_ib)

            @pl.loop(0, CHUNKS, step=2)
            def _chunk2(j):
                issue_gather(j + 1, 1, zb_b, ea_b, sem_zb, sem_eb, sem_ib)
                do_chunk(0, zb_a, ea_a, sem_za, sem_ea)

                @pl.when(j + 2 < CHUNKS)
                def _():
                    issue_idx(j + 2, 0, sem_ia)
                    issue_gather(j + 2, 0, zb_a, ea_a, sem_za, sem_ea,
                                 sem_ia)

                do_chunk(1, zb_b, ea_b, sem_zb, sem_eb)

                @pl.when(j + 3 < CHUNKS)
                def _():
                    issue_idx(j + 3, 1, sem_ib)

            plsc.subcore_barrier()

            pltpu.sync_copy(acc.at[pl.ds(zrow, ROWS_PT)],
                            out_h.at[c, hp, pl.ds(zrow, ROWS_PT)])

            if hp == 0:
                plsc.subcore_barrier()

    return k


def _sc_edge_pass(z, ea_p0, ea_p1, src2d, dst2d, zmaxv, eamaxv, tvec):
    return _build_sc_edge_pass()(z, ea_p0, ea_p1, src2d, dst2d, zmaxv,
                                 eamaxv, tvec)


def _node_proj(x, w, b):
    def body(x_r, w_r, b_r, z_r, m_r):
        i = pl.program_id(0)
        h = jnp.dot(x_r[...], w_r[...],
                    preferred_element_type=jnp.float32) + b_r[...]
        z_r[0] = h
        z_r[1] = jnp.concatenate([h[:, HH:], h[:, :HH]], axis=-1)
        m = jnp.max(h, axis=0, keepdims=True)

        @pl.when(i == 0)
        def _():
            m_r[...] = m

        @pl.when(i != 0)
        def _():
            m_r[...] = jnp.maximum(m_r[...], m)

    return pl.pallas_call(
        body,
        grid=(GRID_N,),
        in_specs=[
            pl.BlockSpec((RB, H), lambda i: (i, 0)),
            pl.BlockSpec((H, H), lambda i: (0, 0)),
            pl.BlockSpec((1, H), lambda i: (0, 0)),
        ],
        out_specs=[
            pl.BlockSpec((2, RB, H), lambda i: (0, i, 0)),
            pl.BlockSpec((1, H), lambda i: (0, 0)),
        ],
        out_shape=[
            jax.ShapeDtypeStruct((2, N, H), jnp.float32),
            jax.ShapeDtypeStruct((1, H), jnp.float32),
        ],
    )(x, w, b)


def _edge_proj(eap, w, b, c0):
    EB = 2048
    grid = EH // EB

    def body(a_lo, a_hi, w_r, b_r, e_r, m_r):
        i = pl.program_id(0)
        wv = w_r[...]
        bv = b_r[...]
        lo = jnp.dot(a_lo[...], wv, preferred_element_type=jnp.float32) + bv
        hi = jnp.dot(a_hi[...], wv, preferred_element_type=jnp.float32) + bv
        e_r[...] = jnp.concatenate(
            [lo[:, c0:c0 + HH], hi[:, c0:c0 + HH]], axis=-1)
        m = jnp.maximum(jnp.max(lo, axis=0, keepdims=True),
                        jnp.max(hi, axis=0, keepdims=True))

        @pl.when(i == 0)
        def _():
            m_r[...] = m

        @pl.when(i != 0)
        def _():
            m_r[...] = jnp.maximum(m_r[...], m)

    return pl.pallas_call(
        body,
        grid=(grid,),
        in_specs=[
            pl.BlockSpec((EB, 16), lambda i: (i, 0)),
            pl.BlockSpec((EB, 16), lambda i: (i + EH // EB, 0)),
            pl.BlockSpec((16, H), lambda i: (0, 0)),
            pl.BlockSpec((1, H), lambda i: (0, 0)),
        ],
        out_specs=[
            pl.BlockSpec((EB, H), lambda i: (i, 0)),
            pl.BlockSpec((1, H), lambda i: (0, 0)),
        ],
        out_shape=[
            jax.ShapeDtypeStruct((EH, H), jnp.float32),
            jax.ShapeDtypeStruct((1, H), jnp.float32),
        ],
    )(eap, eap, w, b)


def _layer_norm_blk(u, g, b):
    m = jnp.mean(u, axis=-1, keepdims=True)
    v = jnp.mean((u - m) ** 2, axis=-1, keepdims=True)
    return (u - m) / jnp.sqrt(v + 1e-5) * g + b


def _combine_mlp(comb, z, h_prev, w1, b1, mg, mb, w2, b2, nlg, nlb):
    def body(lo_r, hi_r, z_r, hp_r, w1_r, b1_r, mg_r, mb_r, w2_r, b2_r,
             g_r, bb_r, h_r, zn_r, m_r):
        lo = lo_r[0, 0]
        hi = hi_r[0, 0]
        den = jnp.concatenate([lo[:, :HH], hi[:, :HH]], axis=-1)
        num = jnp.concatenate([lo[:, HH:], hi[:, HH:]], axis=-1) + EPS * den
        out = num / (den + 1e-16) + z_r[0]
        u = jnp.dot(out, w1_r[...], preferred_element_type=jnp.float32)
        u = _layer_norm_blk(u + b1_r[...], mg_r[...], mb_r[...])
        u = jnp.maximum(u, 0.0)
        v = jnp.dot(u, w2_r[...], preferred_element_type=jnp.float32)
        h = hp_r[...] + v + b2_r[...]
        zn = jnp.maximum(_layer_norm_blk(h, g_r[...], bb_r[...]), 0.0)
        h_r[...] = h
        zn_r[0] = zn
        zn_r[1] = jnp.concatenate([zn[:, HH:], zn[:, :HH]], axis=-1)
        i = pl.program_id(0)
        m = jnp.max(zn, axis=0, keepdims=True)

        @pl.when(i == 0)
        def _():
            m_r[...] = m

        @pl.when(i != 0)
        def _():
            m_r[...] = jnp.maximum(m_r[...], m)

    BPH = NH // RB

    in_specs = [
        pl.BlockSpec((1, 1, RB, H),
                     lambda i: (0, i // BPH, i % BPH, 0)),
        pl.BlockSpec((1, 1, RB, H),
                     lambda i: (1, i // BPH, i % BPH, 0)),
        pl.BlockSpec((1, RB, H), lambda i: (0, i, 0)),
        pl.BlockSpec((RB, H), lambda i: (i, 0)),
        pl.BlockSpec((H, 2 * H), lambda i: (0, 0)),
        pl.BlockSpec((1, 2 * H), lambda i: (0, 0)),
        pl.BlockSpec((1, 2 * H), lambda i: (0, 0)),
        pl.BlockSpec((1, 2 * H), lambda i: (0, 0)),
        pl.BlockSpec((2 * H, H), lambda i: (0, 0)),
        pl.BlockSpec((1, H), lambda i: (0, 0)),
        pl.BlockSpec((1, H), lambda i: (0, 0)),
        pl.BlockSpec((1, H), lambda i: (0, 0)),
    ]
    out_specs = [
        pl.BlockSpec((RB, H), lambda i: (i, 0)),
        pl.BlockSpec((2, RB, H), lambda i: (0, i, 0)),
        pl.BlockSpec((1, H), lambda i: (0, 0)),
    ]
    out_shape = [
        jax.ShapeDtypeStruct((N, H), jnp.float32),
        jax.ShapeDtypeStruct((2, N, H), jnp.float32),
        jax.ShapeDtypeStruct((1, H), jnp.float32),
    ]
    return pl.pallas_call(
        body,
        grid=(GRID_N,),
        in_specs=in_specs,
        out_specs=out_specs,
        out_shape=out_shape,
    )(comb, comb, z, h_prev, w1, b1.reshape(1, -1), mg.reshape(1, -1),
      mb.reshape(1, -1), w2, b2.reshape(1, -1), nlg.reshape(1, -1),
      nlb.reshape(1, -1))


def _head(z, w, b):
    def body(z_r, w_r, b_r, y_r):
        y_r[...] = jnp.dot(z_r[0], w_r[...],
                           preferred_element_type=jnp.float32) + b_r[...]

    return pl.pallas_call(
        body,
        grid=(GRID_N,),
        in_specs=[
            pl.BlockSpec((1, RB, H), lambda i: (0, i, 0)),
            pl.BlockSpec((H, C), lambda i: (0, 0)),
            pl.BlockSpec((1, C), lambda i: (0, 0)),
        ],
        out_specs=pl.BlockSpec((RB, C), lambda i: (i, 0)),
        out_shape=jax.ShapeDtypeStruct((N, C), jnp.float32),
    )(z, w, b.reshape(1, -1))


def kernel(x, edge_index, edge_attr, params):
    src = edge_index[0].astype(jnp.int32)
    dst = edge_index[1].astype(jnp.int32)
    pad = E_PAD - E
    src_p = jnp.concatenate([src, jnp.zeros((pad,), jnp.int32)])
    dst_p = jnp.concatenate([dst, jnp.full((pad,), N, jnp.int32)])
    src2d = jnp.concatenate([src_p[:EH].reshape(-1, B // 2),
                             src_p[EH:].reshape(-1, B // 2)],
                            axis=1).reshape(E_PAD)
    dst2d = jnp.concatenate([dst_p[:EH].reshape(-1, B // 2),
                             dst_p[EH:].reshape(-1, B // 2)],
                            axis=1).reshape(E_PAD)
    eap = jnp.concatenate(
        [edge_attr, jnp.zeros((pad, edge_attr.shape[1]), jnp.float32)])

    ea_p0, eamax = _edge_proj(eap, params['W_edge'],
                              params['b_edge'].reshape(1, -1), 0)
    ea_p1, _ = _edge_proj(eap, params['W_edge'],
                          params['b_edge'].reshape(1, -1), HH)
    eamaxv = eamax.reshape(H)

    z, zmax = _node_proj(x, params['W_node'], params['b_node'].reshape(1, -1))

    ls = params['layers']
    nxt_g = jnp.stack([ls[1]['ln_g'], ls[2]['ln_g'], ls[0]['ln_g']])
    nxt_b = jnp.stack([ls[1]['ln_b'], ls[2]['ln_b'], ls[0]['ln_b']])
    xs = {
        't': jnp.stack([p['t'] for p in ls]),
        'W1': jnp.stack([p['W1'] for p in ls]),
        'b1': jnp.stack([p['b1'] for p in ls]),
        'mg': jnp.stack([p['mln_g'] for p in ls]),
        'mb': jnp.stack([p['mln_b'] for p in ls]),
        'W2': jnp.stack([p['W2'] for p in ls]),
        'b2': jnp.stack([p['b2'] for p in ls]),
        'ng': nxt_g,
        'nb': nxt_b,
    }

    def step(carry, p):
        h, zc, zm = carry
        tvec = jnp.full((16,), p['t'], jnp.float32)
        comb = _sc_edge_pass(zc.reshape(2 * N, H), ea_p0, ea_p1, src2d,
                             dst2d, zm.reshape(H), eamaxv, tvec)
        h2, z2, zm2 = _combine_mlp(comb, zc, h, p['W1'], p['b1'], p['mg'],
                                   p['mb'], p['W2'], p['b2'], p['ng'],
                                   p['nb'])
        return (h2, z2, zm2), None

    h0 = jnp.zeros((N, H), jnp.float32)
    (h, z, zmax), _ = jax.lax.scan(step, (h0, z, zmax), xs)
    return _head(z, params['W_lin'], params['b_lin'])
